# baseline (speedup 1.0000x reference)
import jax
import jax.numpy as jnp
from jax import lax
from jax.experimental import pallas as pl
from jax.experimental.pallas import tpu as pltpu

_B, _T, _H, _NB = 16, 4096, 512, 512
_TC = 512
_NT = _T // _TC


def _pool_body(time_ref, feat_ref, out_ref, cnt_ref):
    t = pl.program_id(1)
    tm = time_ref[0, 0, :]
    oh = (tm[:, None] == lax.broadcasted_iota(jnp.int32, (_TC, _NB), 1)
          ).astype(jnp.bfloat16)
    contrib = lax.dot_general(oh, feat_ref[0].astype(jnp.bfloat16),
                              (((0,), (0,)), ((), ())),
                              preferred_element_type=jnp.float32)
    ones = jnp.ones((8, _TC), dtype=jnp.bfloat16)
    cnt = lax.dot_general(ones, oh, (((1,), (0,)), ((), ())),
                          preferred_element_type=jnp.float32)[0]

    @pl.when(t == 0)
    def _init():
        out_ref[0] = contrib
        cnt_ref[0, 0] = cnt

    @pl.when(t > 0)
    def _acc():
        out_ref[0] += contrib
        cnt_ref[0, 0] += cnt

    @pl.when(t == _NT - 1)
    def _fin():
        out_ref[0] = out_ref[0] / jnp.maximum(cnt_ref[0, 0], 1.0)[:, None]


def kernel(backbone_features, time, temporal_padding_mask):
    marked = jnp.where(temporal_padding_mask, _NB, time).astype(jnp.int32)
    time3 = marked.reshape(_B * _NT, 1, _TC)
    pooled, cnt = pl.pallas_call(
        _pool_body,
        grid=(_B, _NT),
        in_specs=[
            pl.BlockSpec((1, 1, _TC), lambda b, t: (b * _NT + t, 0, 0)),
            pl.BlockSpec((1, _TC, _H), lambda b, t: (b, t, 0)),
        ],
        out_specs=[
            pl.BlockSpec((1, _NB, _H), lambda b, t: (b, 0, 0)),
            pl.BlockSpec((1, 1, _NB), lambda b, t: (b, 0, 0)),
        ],
        out_shape=[
            jax.ShapeDtypeStruct((_B, _NB, _H), jnp.float32),
            jax.ShapeDtypeStruct((_B, 1, _NB), jnp.float32),
        ],
        compiler_params=pltpu.CompilerParams(
            dimension_semantics=("parallel", "arbitrary")),
    )(time3, backbone_features)
    new_padding_mask = cnt.reshape(_B, _NB) == 0.0
    return pooled, new_padding_mask


# speedup vs baseline: 2.1333x; 2.1333x over previous
import jax
import jax.numpy as jnp
from jax import lax
from jax.experimental import pallas as pl
from jax.experimental.pallas import tpu as pltpu

_B, _T, _H, _NB = 16, 4096, 512, 512


def _pool_body(time_ref, feat_ref, out_ref, cnt_ref):
    tm = time_ref[0, 0, :]
    oh = (tm[:, None] == lax.broadcasted_iota(jnp.int32, (_T, _NB), 1)
          ).astype(jnp.bfloat16)
    sums = lax.dot_general(oh, feat_ref[0].astype(jnp.bfloat16),
                           (((0,), (0,)), ((), ())),
                           preferred_element_type=jnp.float32)
    ones = jnp.ones((8, _T), dtype=jnp.bfloat16)
    cnt = lax.dot_general(ones, oh, (((1,), (0,)), ((), ())),
                          preferred_element_type=jnp.float32)[0]
    cnt_ref[0, 0] = cnt
    out_ref[0] = sums / jnp.maximum(cnt, 1.0)[:, None]


def kernel(backbone_features, time, temporal_padding_mask):
    marked = jnp.where(temporal_padding_mask, _NB, time).astype(jnp.int32)
    time3 = marked.reshape(_B, 1, _T)
    pooled, cnt = pl.pallas_call(
        _pool_body,
        grid=(_B,),
        in_specs=[
            pl.BlockSpec((1, 1, _T), lambda b: (b, 0, 0)),
            pl.BlockSpec((1, _T, _H), lambda b: (b, 0, 0)),
        ],
        out_specs=[
            pl.BlockSpec((1, _NB, _H), lambda b: (b, 0, 0)),
            pl.BlockSpec((1, 1, _NB), lambda b: (b, 0, 0)),
        ],
        out_shape=[
            jax.ShapeDtypeStruct((_B, _NB, _H), jnp.float32),
            jax.ShapeDtypeStruct((_B, 1, _NB), jnp.float32),
        ],
        compiler_params=pltpu.CompilerParams(
            dimension_semantics=("arbitrary",)),
    )(time3, backbone_features)
    new_padding_mask = cnt.reshape(_B, _NB) == 0.0
    return pooled, new_padding_mask
